# baseline (speedup 1.0000x reference)
import functools

import jax
import jax.numpy as jnp
from jax import lax
from jax.experimental import pallas as pl
from jax.experimental.pallas import tpu as pltpu
from jax.experimental.pallas import tpu_sc as plsc

MINI_BATCH = 16384
STATE_DIM = 64
NROW = 1000001
NC = 2
NS = 16
NW = NC * NS
B_PER_W = MINI_BATCH // NW
NVEC = B_PER_W // 16

MAIN = 999936
PITCH = 1000064
TAIL_N = NROW - MAIN
TAIL_PAD = 8192
T0 = STATE_DIM * PITCH
FLATP = T0 + TAIL_PAD
WAVE = 8


def _flatten_body(src_ref, tail_ref, dst_ref, sem):
    def _copy(j):
        return pltpu.make_async_copy(
            src_ref.at[j, pl.ds(0, MAIN)],
            dst_ref.at[pl.ds(j * PITCH, MAIN)], sem)

    tail_copy = pltpu.make_async_copy(
        tail_ref, dst_ref.at[pl.ds(T0, TAIL_PAD)], sem)
    tail_copy.start()
    for j in range(WAVE):
        _copy(j).start()
    for w in range(1, STATE_DIM // WAVE):
        for j in range(w * WAVE, (w + 1) * WAVE):
            _copy(j).start()
        for j in range((w - 1) * WAVE, w * WAVE):
            _copy(j).wait()
    for j in range(STATE_DIM - WAVE, STATE_DIM):
        _copy(j).wait()
    tail_copy.wait()


def _flatten(table_t, tail_pad):
    return pl.pallas_call(
        _flatten_body,
        out_shape=jax.ShapeDtypeStruct((FLATP,), jnp.float32),
        in_specs=[pl.BlockSpec(memory_space=pltpu.MemorySpace.HBM),
                  pl.BlockSpec(memory_space=pltpu.MemorySpace.HBM)],
        out_specs=pl.BlockSpec(memory_space=pltpu.MemorySpace.HBM),
        scratch_shapes=[pltpu.SemaphoreType.DMA],
    )(table_t, tail_pad)


def _gather_cols_body(flat_hbm, idx_hbm, out_t, idx_f, off_c, col_c, sem):
    wid = lax.axis_index("s") * NC + lax.axis_index("c")
    base = wid * B_PER_W

    pltpu.sync_copy(idx_hbm.at[pl.ds(base, B_PER_W)], idx_f)

    @pl.loop(0, STATE_DIM)
    def _off(j):
        c_main = j * PITCH
        c_tail = T0 + j * TAIL_N - MAIN
        for k in range(NVEC):
            v = idx_f[pl.ds(k * 16, 16)]
            off_c[j, pl.ds(k * 16, 16)] = jnp.where(
                v < MAIN, v + c_main, v + c_tail)

    copies = []
    for j in range(STATE_DIM):
        copies.append(pltpu.async_copy(
            flat_hbm.at[off_c.at[j]], col_c.at[j], sem))
    for cp in copies:
        cp.wait()

    pltpu.sync_copy(col_c, out_t.at[:, pl.ds(base, B_PER_W)])


def _gather_1d_body(rew_hbm, msk_hbm, act_hbm, idx_hbm,
                    out_rew, out_msk, out_act,
                    idx_f, rew_v, msk_v, act_v, sem):
    wid = lax.axis_index("s") * NC + lax.axis_index("c")
    base = wid * B_PER_W

    pltpu.sync_copy(idx_hbm.at[pl.ds(base, B_PER_W)], idx_f)
    copies = [
        pltpu.async_copy(rew_hbm.at[idx_f], rew_v, sem),
        pltpu.async_copy(msk_hbm.at[idx_f], msk_v, sem),
        pltpu.async_copy(act_hbm.at[idx_f], act_v, sem),
    ]
    for cp in copies:
        cp.wait()
    pltpu.sync_copy(rew_v, out_rew.at[pl.ds(base, B_PER_W)])
    pltpu.sync_copy(msk_v, out_msk.at[pl.ds(base, B_PER_W)])
    pltpu.sync_copy(act_v, out_act.at[pl.ds(base, B_PER_W)])


def _tail(table):
    t = jnp.reshape(jnp.transpose(table[MAIN:, :]), (-1,))
    return jnp.pad(t, (0, TAIL_PAD - STATE_DIM * TAIL_N))


@jax.jit
def kernel(state, next_state, reward, masks, action, idx):
    idx = idx.astype(jnp.int32)
    act_dtype = action.dtype
    mesh = plsc.VectorSubcoreMesh(core_axis_name="c", subcore_axis_name="s")

    gather_cols = pl.kernel(
        _gather_cols_body,
        mesh=mesh,
        compiler_params=pltpu.CompilerParams(use_tc_tiling_on_sc=False),
        out_type=[
            jax.ShapeDtypeStruct((STATE_DIM, MINI_BATCH), jnp.float32),
        ],
        scratch_types=[
            pltpu.VMEM((B_PER_W,), jnp.int32),
            pltpu.VMEM((STATE_DIM, B_PER_W), jnp.int32),
            pltpu.VMEM((STATE_DIM, B_PER_W), jnp.float32),
            pltpu.SemaphoreType.DMA,
        ],
    )
    gather_1d = pl.kernel(
        _gather_1d_body,
        mesh=mesh,
        compiler_params=pltpu.CompilerParams(use_tc_tiling_on_sc=False),
        out_type=[
            jax.ShapeDtypeStruct((MINI_BATCH,), jnp.float32),
            jax.ShapeDtypeStruct((MINI_BATCH,), jnp.float32),
            jax.ShapeDtypeStruct((MINI_BATCH,), act_dtype),
        ],
        scratch_types=[
            pltpu.VMEM((B_PER_W,), jnp.int32),
            pltpu.VMEM((B_PER_W,), jnp.float32),
            pltpu.VMEM((B_PER_W,), jnp.float32),
            pltpu.VMEM((B_PER_W,), act_dtype),
            pltpu.SemaphoreType.DMA,
        ],
    )

    st_flat = _flatten(state.T, _tail(state))
    (out_state_t,) = gather_cols(st_flat, idx)
    out_rew, out_msk, out_act = gather_1d(reward, masks, action, idx)
    nx_flat = _flatten(next_state.T, _tail(next_state))
    (out_next_t,) = gather_cols(nx_flat, idx)
    return (out_state_t.T, out_act, out_rew, out_next_t.T, out_msk)


# speedup vs baseline: 34.3577x; 34.3577x over previous
import functools

import jax
import jax.numpy as jnp
from jax import lax
from jax.experimental import pallas as pl
from jax.experimental.pallas import tpu as pltpu
from jax.experimental.pallas import tpu_sc as plsc

MINI_BATCH = 16384
STATE_DIM = 64
NROW = 1000001
NC = 2
NS = 16
NW = NC * NS
B_PER_W = MINI_BATCH // NW
NVEC = B_PER_W // 16

MAIN = 999936
PITCH = 1000064
TAIL_N = NROW - MAIN
TAIL_PAD = 8192
T0 = STATE_DIM * PITCH
FLATP = T0 + TAIL_PAD
WAVE = 8


FW = 142848
FC = MAIN // FW


def _flatten_body(x_ref, tail_ref, dst_ref, sem, tsem):
    g = pl.program_id(0)
    c = pl.program_id(1)
    step = g * FC + c
    last = 8 * FC - 1

    tail_copy = pltpu.make_async_copy(
        tail_ref, dst_ref.at[pl.ds(T0, TAIL_PAD)], tsem)

    @pl.when(step == 0)
    def _():
        tail_copy.start()

    for m in range(8):
        row = g * 8 + m
        pltpu.make_async_copy(
            x_ref.at[m],
            dst_ref.at[pl.ds(row * PITCH + c * FW, FW)], sem).start()

    for m in range(8):
        pltpu.make_async_copy(
            x_ref.at[m], dst_ref.at[pl.ds(0, FW)], sem).wait()

    @pl.when(step == last)
    def _():
        tail_copy.wait()


def _flatten(table_t, tail_pad):
    return pl.pallas_call(
        _flatten_body,
        grid=(8, FC),
        out_shape=jax.ShapeDtypeStruct((FLATP,), jnp.float32),
        in_specs=[pl.BlockSpec((8, FW), lambda g, c: (g, c)),
                  pl.BlockSpec(memory_space=pltpu.MemorySpace.HBM)],
        out_specs=pl.BlockSpec(memory_space=pltpu.MemorySpace.HBM),
        scratch_shapes=[pltpu.SemaphoreType.DMA, pltpu.SemaphoreType.DMA],
    )(table_t, tail_pad)


def _gather_cols_body(flat_hbm, idx_hbm, out_t, idx_f, off_c, col_c, sem):
    wid = lax.axis_index("s") * NC + lax.axis_index("c")
    base = wid * B_PER_W

    pltpu.sync_copy(idx_hbm.at[pl.ds(base, B_PER_W)], idx_f)

    @pl.loop(0, STATE_DIM)
    def _off(j):
        c_main = j * PITCH
        c_tail = T0 + j * TAIL_N - MAIN
        for k in range(NVEC):
            v = idx_f[pl.ds(k * 16, 16)]
            off_c[j, pl.ds(k * 16, 16)] = jnp.where(
                v < MAIN, v + c_main, v + c_tail)

    copies = []
    for j in range(STATE_DIM):
        copies.append(pltpu.async_copy(
            flat_hbm.at[off_c.at[j]], col_c.at[j], sem))
    for cp in copies:
        cp.wait()

    pltpu.sync_copy(col_c, out_t.at[:, pl.ds(base, B_PER_W)])


def _gather_1d_body(rew_hbm, msk_hbm, act_hbm, idx_hbm,
                    out_rew, out_msk, out_act,
                    idx_f, rew_v, msk_v, act_v, sem):
    wid = lax.axis_index("s") * NC + lax.axis_index("c")
    base = wid * B_PER_W

    pltpu.sync_copy(idx_hbm.at[pl.ds(base, B_PER_W)], idx_f)
    copies = [
        pltpu.async_copy(rew_hbm.at[idx_f], rew_v, sem),
        pltpu.async_copy(msk_hbm.at[idx_f], msk_v, sem),
        pltpu.async_copy(act_hbm.at[idx_f], act_v, sem),
    ]
    for cp in copies:
        cp.wait()
    pltpu.sync_copy(rew_v, out_rew.at[pl.ds(base, B_PER_W)])
    pltpu.sync_copy(msk_v, out_msk.at[pl.ds(base, B_PER_W)])
    pltpu.sync_copy(act_v, out_act.at[pl.ds(base, B_PER_W)])


def _tail(table):
    t = jnp.reshape(jnp.transpose(table[MAIN:, :]), (-1,))
    return jnp.pad(t, (0, TAIL_PAD - STATE_DIM * TAIL_N))


@jax.jit
def kernel(state, next_state, reward, masks, action, idx):
    idx = idx.astype(jnp.int32)
    act_dtype = action.dtype
    mesh = plsc.VectorSubcoreMesh(core_axis_name="c", subcore_axis_name="s")

    gather_cols = pl.kernel(
        _gather_cols_body,
        mesh=mesh,
        compiler_params=pltpu.CompilerParams(use_tc_tiling_on_sc=False),
        out_type=[
            jax.ShapeDtypeStruct((STATE_DIM, MINI_BATCH), jnp.float32),
        ],
        scratch_types=[
            pltpu.VMEM((B_PER_W,), jnp.int32),
            pltpu.VMEM((STATE_DIM, B_PER_W), jnp.int32),
            pltpu.VMEM((STATE_DIM, B_PER_W), jnp.float32),
            pltpu.SemaphoreType.DMA,
        ],
    )
    gather_1d = pl.kernel(
        _gather_1d_body,
        mesh=mesh,
        compiler_params=pltpu.CompilerParams(use_tc_tiling_on_sc=False),
        out_type=[
            jax.ShapeDtypeStruct((MINI_BATCH,), jnp.float32),
            jax.ShapeDtypeStruct((MINI_BATCH,), jnp.float32),
            jax.ShapeDtypeStruct((MINI_BATCH,), act_dtype),
        ],
        scratch_types=[
            pltpu.VMEM((B_PER_W,), jnp.int32),
            pltpu.VMEM((B_PER_W,), jnp.float32),
            pltpu.VMEM((B_PER_W,), jnp.float32),
            pltpu.VMEM((B_PER_W,), act_dtype),
            pltpu.SemaphoreType.DMA,
        ],
    )

    st_flat = _flatten(state.T, _tail(state))
    (out_state_t,) = gather_cols(st_flat, idx)
    out_rew, out_msk, out_act = gather_1d(reward, masks, action, idx)
    nx_flat = _flatten(next_state.T, _tail(next_state))
    (out_next_t,) = gather_cols(nx_flat, idx)
    return (out_state_t.T, out_act, out_rew, out_next_t.T, out_msk)
